# baseline (speedup 1.0000x reference)
import functools

import jax
import jax.numpy as jnp
from jax import lax
from jax.experimental import pallas as pl
from jax.experimental.pallas import tpu as pltpu
from jax.experimental.pallas import tpu_sc as plsc

B = 16
AB = 20000
V = 12000
NW = 32
NJ = 3
G = 128
N = NJ * G
VP = NW * N


def _sc_body(scores_hbm, deltas_hbm, anch_hbm, idx_hbm, out_hbm,
             idx_v, sr_v, sub_v, big_v, s_v,
             ax_v, ay_v, aw_v, ah_v, o_v, sem):
    wid = lax.axis_index("s") * 2 + lax.axis_index("c")
    base = wid * N
    iota = lax.iota(jnp.int32, 16)
    two = jnp.full((16,), 2, jnp.int32)
    c0 = jnp.zeros((16,), jnp.int32)
    c1 = jnp.full((16,), 1, jnp.int32)
    c2 = jnp.full((16,), 2, jnp.int32)
    c3 = jnp.full((16,), 3, jnp.int32)
    c4 = jnp.full((16,), 4, jnp.int32)

    for j in range(NJ):
        pltpu.sync_copy(idx_hbm.at[pl.ds(base + G * j, G)], idx_v.at[j])
        for i in range(G // 16):
            raw = idx_v[j, pl.ds(i * 16, 16)]
            sr_v[j, pl.ds(i * 16, 16)] = lax.shift_right_logical(raw, two)
            sub_v[j, pl.ds(i * 16, 16)] = lax.shift_left(
                jnp.bitwise_and(raw, c3), two)

    for j in range(NJ):
        pltpu.async_copy(anch_hbm.at[sr_v.at[j]], big_v.at[j], sem).wait()
        for i in range(G // 16):
            pos = iota + i * 16
            sub = sub_v[j, pl.ds(i * 16, 16)]
            ax_v[j, pl.ds(i * 16, 16)] = plsc.load_gather(big_v.at[j], [pos, sub])
            ay_v[j, pl.ds(i * 16, 16)] = plsc.load_gather(big_v.at[j], [pos, sub + c1])
            aw_v[j, pl.ds(i * 16, 16)] = plsc.load_gather(big_v.at[j], [pos, sub + c2])
            ah_v[j, pl.ds(i * 16, 16)] = plsc.load_gather(big_v.at[j], [pos, sub + c3])

    def batch_body(b, carry):
        cps = []
        for j in range(NJ):
            cps.append(pltpu.async_copy(
                scores_hbm.at[b].at[idx_v.at[j]], s_v.at[j], sem))
            cps.append(pltpu.async_copy(
                deltas_hbm.at[b].at[sr_v.at[j]], big_v.at[j], sem))
        for cp in cps:
            cp.wait()
        for j in range(NJ):
            bj = big_v.at[j]
            oj = o_v.at[j]
            for i in range(G // 16):
                pos = iota + (i * 16)
                sub = sub_v[j, pl.ds(i * 16, 16)]
                dx = plsc.load_gather(bj, [pos, sub])
                dy = plsc.load_gather(bj, [pos, sub + c1])
                dw = plsc.load_gather(bj, [pos, sub + c2])
                dh = plsc.load_gather(bj, [pos, sub + c3])
                ax = ax_v[j, pl.ds(i * 16, 16)]
                ay = ay_v[j, pl.ds(i * 16, 16)]
                aw = aw_v[j, pl.ds(i * 16, 16)]
                ah = ah_v[j, pl.ds(i * 16, 16)]
                s = s_v[j, pl.ds(i * 16, 16)]
                rx = ax + dx * aw
                ry = ay + dy * ah
                rw = aw * jnp.exp(dw)
                rh = ah * jnp.exp(dh)
                plsc.store_scatter(oj, [pos, c0], s)
                plsc.store_scatter(oj, [pos, c1], rx)
                plsc.store_scatter(oj, [pos, c2], ry)
                plsc.store_scatter(oj, [pos, c3], rw)
                plsc.store_scatter(oj, [pos, c4], rh)
            pltpu.sync_copy(oj, out_hbm.at[b].at[pl.ds(base + G * j, G)])
        return carry

    lax.fori_loop(0, B, batch_body, 0)


@functools.partial(
    pl.kernel,
    out_type=jax.ShapeDtypeStruct((B, VP, 5), jnp.float32),
    mesh=plsc.VectorSubcoreMesh(core_axis_name="c", subcore_axis_name="s"),
    compiler_params=pltpu.CompilerParams(
        needs_layout_passes=False, use_tc_tiling_on_sc=False),
    scratch_types=[
        pltpu.VMEM((NJ, G), jnp.int32),
        pltpu.VMEM((NJ, G), jnp.int32),
        pltpu.VMEM((NJ, G), jnp.int32),
        pltpu.VMEM((NJ, G, 16), jnp.float32),
        pltpu.VMEM((NJ, G), jnp.float32),
        pltpu.VMEM((NJ, G), jnp.float32),
        pltpu.VMEM((NJ, G), jnp.float32),
        pltpu.VMEM((NJ, G), jnp.float32),
        pltpu.VMEM((NJ, G), jnp.float32),
        pltpu.VMEM((NJ, G, 5), jnp.float32),
        pltpu.SemaphoreType.DMA,
    ],
)
def _apply_deltas_sc(scores_hbm, deltas_hbm, anch_hbm, idx_hbm, out_hbm,
                     *refs):
    _sc_body(scores_hbm, deltas_hbm, anch_hbm, idx_hbm, out_hbm, *refs)


def kernel(scores, deltas, anchor_boxes, valid_indices):
    idx = valid_indices.astype(jnp.int32)
    idx = jnp.concatenate([idx, jnp.zeros((VP - V,), jnp.int32)])
    deltas_sr = deltas.reshape(B, AB // 4, 16)
    anch_sr = anchor_boxes.reshape(AB // 4, 16)
    out = _apply_deltas_sc(scores, deltas_sr, anch_sr, idx)
    return out[:, :V, :]


# speedup vs baseline: 1.2170x; 1.2170x over previous
import functools

import jax
import jax.numpy as jnp
from jax import lax
from jax.experimental import pallas as pl
from jax.experimental.pallas import tpu as pltpu
from jax.experimental.pallas import tpu_sc as plsc

B = 16
AB = 20000
V = 12000
NW = 32
NJ = 3
G = 128
N = NJ * G
VP = NW * N


def _sc_body(scores_hbm, deltas_hbm, anch_hbm, idx_hbm, out_hbm,
             idx_v, sr_v, sub_v, big_v, s_v,
             ax_v, ay_v, aw_v, ah_v, o_v, sem):
    wid = lax.axis_index("s") * 2 + lax.axis_index("c")
    base = wid * N
    iota = lax.iota(jnp.int32, 16)
    two = jnp.full((16,), 2, jnp.int32)
    c0 = jnp.zeros((16,), jnp.int32)
    c1 = jnp.full((16,), 1, jnp.int32)
    c2 = jnp.full((16,), 2, jnp.int32)
    c3 = jnp.full((16,), 3, jnp.int32)
    c4 = jnp.full((16,), 4, jnp.int32)

    for j in range(NJ):
        pltpu.sync_copy(idx_hbm.at[pl.ds(base + G * j, G)], idx_v.at[j])
        for i in range(G // 16):
            raw = idx_v[j, pl.ds(i * 16, 16)]
            sr_v[j, pl.ds(i * 16, 16)] = lax.shift_right_logical(raw, two)
            sub_v[j, pl.ds(i * 16, 16)] = lax.shift_left(
                jnp.bitwise_and(raw, c3), two)

    for j in range(NJ):
        pltpu.async_copy(anch_hbm.at[sr_v.at[j]], big_v.at[j], sem).wait()
        for i in range(G // 16):
            pos = iota + i * 16
            sub = sub_v[j, pl.ds(i * 16, 16)]
            ax_v[j, pl.ds(i * 16, 16)] = plsc.load_gather(big_v.at[j], [pos, sub])
            ay_v[j, pl.ds(i * 16, 16)] = plsc.load_gather(big_v.at[j], [pos, sub + c1])
            aw_v[j, pl.ds(i * 16, 16)] = plsc.load_gather(big_v.at[j], [pos, sub + c2])
            ah_v[j, pl.ds(i * 16, 16)] = plsc.load_gather(big_v.at[j], [pos, sub + c3])

    is_last = wid == NW - 1
    not_last = wid != NW - 1

    def batch_body(b, carry):
        cps = []
        for j in range(NJ):
            cps.append(pltpu.async_copy(
                scores_hbm.at[b].at[idx_v.at[j]], s_v.at[j], sem))
            cps.append(pltpu.async_copy(
                deltas_hbm.at[b].at[sr_v.at[j]], big_v.at[j], sem))
        for cp in cps:
            cp.wait()
        for j in range(NJ):
            bj = big_v.at[j]
            oj = o_v.at[j]
            for i in range(G // 16):
                pos = iota + (i * 16)
                sub = sub_v[j, pl.ds(i * 16, 16)]
                dx = plsc.load_gather(bj, [pos, sub])
                dy = plsc.load_gather(bj, [pos, sub + c1])
                dw = plsc.load_gather(bj, [pos, sub + c2])
                dh = plsc.load_gather(bj, [pos, sub + c3])
                ax = ax_v[j, pl.ds(i * 16, 16)]
                ay = ay_v[j, pl.ds(i * 16, 16)]
                aw = aw_v[j, pl.ds(i * 16, 16)]
                ah = ah_v[j, pl.ds(i * 16, 16)]
                s = s_v[j, pl.ds(i * 16, 16)]
                rx = ax + dx * aw
                ry = ay + dy * ah
                rw = aw * jnp.exp(dw)
                rh = ah * jnp.exp(dh)
                plsc.store_scatter(oj, [pos, c0], s)
                plsc.store_scatter(oj, [pos, c1], rx)
                plsc.store_scatter(oj, [pos, c2], ry)
                plsc.store_scatter(oj, [pos, c3], rw)
                plsc.store_scatter(oj, [pos, c4], rh)
            @pl.when(not_last)
            def _():
                pltpu.sync_copy(oj, out_hbm.at[b].at[pl.ds(base + G * j, G)])
            if j == 0:
                @pl.when(is_last)
                def _():
                    pltpu.sync_copy(
                        oj.at[pl.ds(0, V - (NW - 1) * N)],
                        out_hbm.at[b].at[pl.ds(base, V - (NW - 1) * N)])
        return carry

    lax.fori_loop(0, B, batch_body, 0)


@functools.partial(
    pl.kernel,
    out_type=jax.ShapeDtypeStruct((B, V, 5), jnp.float32),
    mesh=plsc.VectorSubcoreMesh(core_axis_name="c", subcore_axis_name="s"),
    compiler_params=pltpu.CompilerParams(
        needs_layout_passes=False, use_tc_tiling_on_sc=False),
    scratch_types=[
        pltpu.VMEM((NJ, G), jnp.int32),
        pltpu.VMEM((NJ, G), jnp.int32),
        pltpu.VMEM((NJ, G), jnp.int32),
        pltpu.VMEM((NJ, G, 16), jnp.float32),
        pltpu.VMEM((NJ, G), jnp.float32),
        pltpu.VMEM((NJ, G), jnp.float32),
        pltpu.VMEM((NJ, G), jnp.float32),
        pltpu.VMEM((NJ, G), jnp.float32),
        pltpu.VMEM((NJ, G), jnp.float32),
        pltpu.VMEM((NJ, G, 5), jnp.float32),
        pltpu.SemaphoreType.DMA,
    ],
)
def _apply_deltas_sc(scores_hbm, deltas_hbm, anch_hbm, idx_hbm, out_hbm,
                     *refs):
    _sc_body(scores_hbm, deltas_hbm, anch_hbm, idx_hbm, out_hbm, *refs)


def kernel(scores, deltas, anchor_boxes, valid_indices):
    idx = valid_indices.astype(jnp.int32)
    idx = jnp.concatenate([idx, jnp.zeros((VP - V,), jnp.int32)])
    deltas_sr = deltas.reshape(B, AB // 4, 16)
    anch_sr = anchor_boxes.reshape(AB // 4, 16)
    return _apply_deltas_sc(scores, deltas_sr, anch_sr, idx)


# speedup vs baseline: 3.2741x; 2.6903x over previous
import functools

import jax
import jax.numpy as jnp
from jax import lax
from jax.experimental import pallas as pl
from jax.experimental.pallas import tpu as pltpu
from jax.experimental.pallas import tpu_sc as plsc

B = 16
AB = 20000
V = 12000
NW = 32
NJ = 3
G = 128
N = NJ * G
VP = NW * N
LASTN = V - (NW - 1) * N


def _sc_body(scores_hbm, deltas_hbm, anch_hbm, idx_hbm, out_hbm,
             idx_v, s_v, dx_v, dy_v, dw_v, dh_v,
             ax_v, ay_v, aw_v, ah_v, ox_v, oy_v, ow_v, oh_v, sem):
    wid = lax.axis_index("s") * 2 + lax.axis_index("c")
    base = wid * N
    is_last = wid == NW - 1
    not_last = wid != NW - 1

    for j in range(NJ):
        pltpu.sync_copy(idx_hbm.at[pl.ds(base + G * j, G)], idx_v.at[j])
    cps = []
    for j in range(NJ):
        ij = idx_v.at[j]
        cps.append(pltpu.async_copy(anch_hbm.at[0].at[ij], ax_v.at[j], sem))
        cps.append(pltpu.async_copy(anch_hbm.at[1].at[ij], ay_v.at[j], sem))
        cps.append(pltpu.async_copy(anch_hbm.at[2].at[ij], aw_v.at[j], sem))
        cps.append(pltpu.async_copy(anch_hbm.at[3].at[ij], ah_v.at[j], sem))
    for cp in cps:
        cp.wait()

    def store_plane(c, b, j, src):
        @pl.when(not_last)
        def _():
            pltpu.sync_copy(src, out_hbm.at[c].at[b].at[pl.ds(base + G * j, G)])
        if j == 0:
            @pl.when(is_last)
            def _():
                pltpu.sync_copy(src.at[pl.ds(0, LASTN)],
                                out_hbm.at[c].at[b].at[pl.ds(base, LASTN)])

    def batch_body(b, carry):
        row = b * 4
        cps = []
        for j in range(NJ):
            ij = idx_v.at[j]
            cps.append(pltpu.async_copy(
                scores_hbm.at[b].at[ij], s_v.at[j], sem))
            cps.append(pltpu.async_copy(
                deltas_hbm.at[row].at[ij], dx_v.at[j], sem))
            cps.append(pltpu.async_copy(
                deltas_hbm.at[row + 1].at[ij], dy_v.at[j], sem))
            cps.append(pltpu.async_copy(
                deltas_hbm.at[row + 2].at[ij], dw_v.at[j], sem))
            cps.append(pltpu.async_copy(
                deltas_hbm.at[row + 3].at[ij], dh_v.at[j], sem))
        for cp in cps:
            cp.wait()
        for j in range(NJ):
            for i in range(G // 16):
                sl = pl.ds(i * 16, 16)
                dx = dx_v[j, sl]
                dy = dy_v[j, sl]
                dw = dw_v[j, sl]
                dh = dh_v[j, sl]
                ax = ax_v[j, sl]
                ay = ay_v[j, sl]
                aw = aw_v[j, sl]
                ah = ah_v[j, sl]
                ox_v[j, sl] = ax + dx * aw
                oy_v[j, sl] = ay + dy * ah
                ow_v[j, sl] = aw * jnp.exp(dw)
                oh_v[j, sl] = ah * jnp.exp(dh)
            store_plane(0, b, j, s_v.at[j])
            store_plane(1, b, j, ox_v.at[j])
            store_plane(2, b, j, oy_v.at[j])
            store_plane(3, b, j, ow_v.at[j])
            store_plane(4, b, j, oh_v.at[j])
        return carry

    lax.fori_loop(0, B, batch_body, 0)


_PLANE = pltpu.VMEM((NJ, G), jnp.float32)


@functools.partial(
    pl.kernel,
    out_type=jax.ShapeDtypeStruct((5, B, V), jnp.float32),
    mesh=plsc.VectorSubcoreMesh(core_axis_name="c", subcore_axis_name="s"),
    compiler_params=pltpu.CompilerParams(
        needs_layout_passes=False, use_tc_tiling_on_sc=False),
    scratch_types=[
        pltpu.VMEM((NJ, G), jnp.int32),
        _PLANE, _PLANE, _PLANE, _PLANE, _PLANE,
        _PLANE, _PLANE, _PLANE, _PLANE,
        _PLANE, _PLANE, _PLANE, _PLANE,
        pltpu.SemaphoreType.DMA,
    ],
)
def _apply_deltas_sc(scores_hbm, deltas_hbm, anch_hbm, idx_hbm, out_hbm,
                     *refs):
    _sc_body(scores_hbm, deltas_hbm, anch_hbm, idx_hbm, out_hbm, *refs)


def kernel(scores, deltas, anchor_boxes, valid_indices):
    idx = valid_indices.astype(jnp.int32)
    idx = jnp.concatenate([idx, jnp.zeros((VP - V,), jnp.int32)])
    deltas_t = jnp.transpose(deltas, (0, 2, 1)).reshape(B * 4, AB)
    anch_t = jnp.transpose(anchor_boxes, (1, 0))
    out = _apply_deltas_sc(scores, deltas_t, anch_t, idx)
    return jnp.transpose(out, (1, 2, 0))


# speedup vs baseline: 3.9180x; 1.1967x over previous
import functools

import jax
import jax.numpy as jnp
from jax import lax
from jax.experimental import pallas as pl
from jax.experimental.pallas import tpu as pltpu
from jax.experimental.pallas import tpu_sc as plsc

B = 16
AB = 20000
V = 12000
NW = 32
NJ = 3
G = 128
N = NJ * G
VP = NW * N
LASTN = V - (NW - 1) * N


def _sc_body(scores_hbm, deltas_hbm, anch_hbm, idx_hbm, out_hbm,
             idx_v, ax_v, ay_v, aw_v, ah_v,
             s_v, dx_v, dy_v, dw_v, dh_v,
             os_v, ox_v, oy_v, ow_v, oh_v,
             sem_ga, sem_gb, sem_sa, sem_sb):
    wid = lax.axis_index("s") * 2 + lax.axis_index("c")
    base = wid * N
    is_last = wid == NW - 1
    not_last = wid != NW - 1

    for j in range(NJ):
        pltpu.sync_copy(idx_hbm.at[pl.ds(base + G * j, G)], idx_v.at[j])
    cps = []
    for j in range(NJ):
        ij = idx_v.at[j]
        cps.append(pltpu.async_copy(anch_hbm.at[0].at[ij], ax_v.at[j], sem_ga))
        cps.append(pltpu.async_copy(anch_hbm.at[1].at[ij], ay_v.at[j], sem_ga))
        cps.append(pltpu.async_copy(anch_hbm.at[2].at[ij], aw_v.at[j], sem_ga))
        cps.append(pltpu.async_copy(anch_hbm.at[3].at[ij], ah_v.at[j], sem_ga))
    for cp in cps:
        cp.wait()

    def gather_descs(b, p, sem):
        row = b * 4
        ds = []
        for j in range(NJ):
            ij = idx_v.at[j]
            ds.append(pltpu.make_async_copy(
                scores_hbm.at[b].at[ij], s_v.at[p].at[j], sem))
            ds.append(pltpu.make_async_copy(
                deltas_hbm.at[row].at[ij], dx_v.at[p].at[j], sem))
            ds.append(pltpu.make_async_copy(
                deltas_hbm.at[row + 1].at[ij], dy_v.at[p].at[j], sem))
            ds.append(pltpu.make_async_copy(
                deltas_hbm.at[row + 2].at[ij], dw_v.at[p].at[j], sem))
            ds.append(pltpu.make_async_copy(
                deltas_hbm.at[row + 3].at[ij], dh_v.at[p].at[j], sem))
        return ds

    def fire_gathers(b, p, sem):
        for d in gather_descs(b, p, sem):
            d.start()

    def wait_gathers(b, p, sem):
        for d in gather_descs(b, p, sem):
            d.wait()

    def store_descs(b, p, sem):
        ds = []
        for c, buf in ((0, os_v), (1, ox_v), (2, oy_v), (3, ow_v), (4, oh_v)):
            for j in range(NJ):
                ds.append((j, pltpu.make_async_copy(
                    buf.at[p].at[j],
                    out_hbm.at[c].at[b].at[pl.ds(base + G * j, G)], sem)))
            ds.append((-1, pltpu.make_async_copy(
                buf.at[p].at[0].at[pl.ds(0, LASTN)],
                out_hbm.at[c].at[b].at[pl.ds(base, LASTN)], sem)))
        return ds

    def fire_stores(b, p, sem):
        for j, d in store_descs(b, p, sem):
            if j >= 0:
                @pl.when(not_last)
                def _():
                    d.start()
            else:
                @pl.when(is_last)
                def _():
                    d.start()

    def drain_stores(b, p, sem):
        for j, d in store_descs(b, p, sem):
            if j >= 0:
                @pl.when(not_last)
                def _():
                    d.wait()
            else:
                @pl.when(is_last)
                def _():
                    d.wait()

    def compute(p):
        for j in range(NJ):
            for i in range(G // 16):
                sl = pl.ds(i * 16, 16)
                dx = dx_v[p, j, sl]
                dy = dy_v[p, j, sl]
                dw = dw_v[p, j, sl]
                dh = dh_v[p, j, sl]
                ax = ax_v[j, sl]
                ay = ay_v[j, sl]
                aw = aw_v[j, sl]
                ah = ah_v[j, sl]
                os_v[p, j, sl] = s_v[p, j, sl]
                ox_v[p, j, sl] = ax + dx * aw
                oy_v[p, j, sl] = ay + dy * ah
                ow_v[p, j, sl] = aw * jnp.exp(dw)
                oh_v[p, j, sl] = ah * jnp.exp(dh)

    fire_gathers(0, 0, sem_ga)

    def body(t, carry):
        b0 = 2 * t
        b1 = 2 * t + 1
        fire_gathers(b1, 1, sem_gb)
        wait_gathers(b0, 0, sem_ga)

        @pl.when(t > 0)
        def _():
            drain_stores(b0 - 2, 0, sem_sa)
        compute(0)
        fire_stores(b0, 0, sem_sa)

        @pl.when(t < (B // 2) - 1)
        def _():
            fire_gathers(b0 + 2, 0, sem_ga)
        wait_gathers(b1, 1, sem_gb)

        @pl.when(t > 0)
        def _():
            drain_stores(b1 - 2, 1, sem_sb)
        compute(1)
        fire_stores(b1, 1, sem_sb)
        return carry

    lax.fori_loop(0, B // 2, body, 0)
    drain_stores(B - 2, 0, sem_sa)
    drain_stores(B - 1, 1, sem_sb)


_PLANE = pltpu.VMEM((NJ, G), jnp.float32)
_PLANE2 = pltpu.VMEM((2, NJ, G), jnp.float32)


@functools.partial(
    pl.kernel,
    out_type=jax.ShapeDtypeStruct((5, B, V), jnp.float32),
    mesh=plsc.VectorSubcoreMesh(core_axis_name="c", subcore_axis_name="s"),
    compiler_params=pltpu.CompilerParams(
        needs_layout_passes=False, use_tc_tiling_on_sc=False),
    scratch_types=[
        pltpu.VMEM((NJ, G), jnp.int32),
        _PLANE, _PLANE, _PLANE, _PLANE,
        _PLANE2, _PLANE2, _PLANE2, _PLANE2, _PLANE2,
        _PLANE2, _PLANE2, _PLANE2, _PLANE2, _PLANE2,
        pltpu.SemaphoreType.DMA,
        pltpu.SemaphoreType.DMA,
        pltpu.SemaphoreType.DMA,
        pltpu.SemaphoreType.DMA,
    ],
)
def _apply_deltas_sc(scores_hbm, deltas_hbm, anch_hbm, idx_hbm, out_hbm,
                     *refs):
    _sc_body(scores_hbm, deltas_hbm, anch_hbm, idx_hbm, out_hbm, *refs)


def kernel(scores, deltas, anchor_boxes, valid_indices):
    idx = valid_indices.astype(jnp.int32)
    idx = jnp.concatenate([idx, jnp.zeros((VP - V,), jnp.int32)])
    deltas_t = jnp.transpose(deltas, (0, 2, 1)).reshape(B * 4, AB)
    anch_t = jnp.transpose(anchor_boxes, (1, 0))
    out = _apply_deltas_sc(scores, deltas_t, anch_t, idx)
    return jnp.transpose(out, (1, 2, 0))


# speedup vs baseline: 5.6752x; 1.4485x over previous
import functools

import jax
import jax.numpy as jnp
from jax import lax
from jax.experimental import pallas as pl
from jax.experimental.pallas import tpu as pltpu
from jax.experimental.pallas import tpu_sc as plsc

B = 16
AB = 20000
V = 12000
NW = 32
NJ = 3
G = 128
N = NJ * G
VP = NW * N
LASTN = V - (NW - 1) * N
W = 1024


def _sc_body(scores_hbm, deltas_hbm, anch_hbm, idx_hbm, out_hbm,
             idx_v, loc_v, ax_v, ay_v, aw_v, ah_v,
             s_v, dx_v, dy_v, dw_v, dh_v,
             ws_v, wdx_v, wdy_v, wdw_v, wdh_v,
             os_v, ox_v, oy_v, ow_v, oh_v,
             sem_ga, sem_gb, sem_sa, sem_sb):
    wid = lax.axis_index("s") * 2 + lax.axis_index("c")
    base = wid * N
    is_last = wid == NW - 1
    not_last = wid != NW - 1

    for j in range(NJ):
        pltpu.sync_copy(idx_hbm.at[pl.ds(base + G * j, G)], idx_v.at[j])
    cps = []
    for j in range(NJ):
        ij = idx_v.at[j]
        cps.append(pltpu.async_copy(anch_hbm.at[0].at[ij], ax_v.at[j], sem_ga))
        cps.append(pltpu.async_copy(anch_hbm.at[1].at[ij], ay_v.at[j], sem_ga))
        cps.append(pltpu.async_copy(anch_hbm.at[2].at[ij], aw_v.at[j], sem_ga))
        cps.append(pltpu.async_copy(anch_hbm.at[3].at[ij], ah_v.at[j], sem_ga))
    for cp in cps:
        cp.wait()

    lo = lax.reduce_min(idx_v[0, pl.ds(0, 16)], (0,))
    hi = lax.reduce_max(idx_v[NJ - 1, pl.ds(G - 16, 16)], (0,))
    lo_al = pl.multiple_of(
        jnp.minimum((lo >> 3) << 3, jnp.int32(AB - W)), 8)
    span_ok = (hi - lo_al) < W
    span_bad = jnp.logical_not(span_ok)
    for j in range(NJ):
        for i in range(G // 16):
            sl = pl.ds(i * 16, 16)
            loc_v[j, sl] = idx_v[j, sl] - lo_al

    def fast_descs(b, p, sem):
        row = b * 4
        win = pl.ds(lo_al, W)
        return [
            pltpu.make_async_copy(scores_hbm.at[b].at[win], ws_v.at[p], sem),
            pltpu.make_async_copy(deltas_hbm.at[row].at[win], wdx_v.at[p], sem),
            pltpu.make_async_copy(deltas_hbm.at[row + 1].at[win], wdy_v.at[p], sem),
            pltpu.make_async_copy(deltas_hbm.at[row + 2].at[win], wdw_v.at[p], sem),
            pltpu.make_async_copy(deltas_hbm.at[row + 3].at[win], wdh_v.at[p], sem),
        ]

    def slow_descs(b, p, sem):
        row = b * 4
        ds = []
        for j in range(NJ):
            ij = idx_v.at[j]
            ds.append(pltpu.make_async_copy(
                scores_hbm.at[b].at[ij], s_v.at[p].at[j], sem))
            ds.append(pltpu.make_async_copy(
                deltas_hbm.at[row].at[ij], dx_v.at[p].at[j], sem))
            ds.append(pltpu.make_async_copy(
                deltas_hbm.at[row + 1].at[ij], dy_v.at[p].at[j], sem))
            ds.append(pltpu.make_async_copy(
                deltas_hbm.at[row + 2].at[ij], dw_v.at[p].at[j], sem))
            ds.append(pltpu.make_async_copy(
                deltas_hbm.at[row + 3].at[ij], dh_v.at[p].at[j], sem))
        return ds

    def fire_gathers(b, p, sem):
        @pl.when(span_ok)
        def _():
            for d in fast_descs(b, p, sem):
                d.start()

        @pl.when(span_bad)
        def _():
            for d in slow_descs(b, p, sem):
                d.start()

    def wait_gathers(b, p, sem):
        @pl.when(span_ok)
        def _():
            for d in fast_descs(b, p, sem):
                d.wait()

        @pl.when(span_bad)
        def _():
            for d in slow_descs(b, p, sem):
                d.wait()

    def store_descs(b, p, sem):
        ds = []
        for c, buf in ((0, os_v), (1, ox_v), (2, oy_v), (3, ow_v), (4, oh_v)):
            for j in range(NJ):
                ds.append((j, pltpu.make_async_copy(
                    buf.at[p].at[j],
                    out_hbm.at[c].at[b].at[pl.ds(base + G * j, G)], sem)))
            ds.append((-1, pltpu.make_async_copy(
                buf.at[p].at[0].at[pl.ds(0, LASTN)],
                out_hbm.at[c].at[b].at[pl.ds(base, LASTN)], sem)))
        return ds

    def fire_stores(b, p, sem):
        for j, d in store_descs(b, p, sem):
            if j >= 0:
                @pl.when(not_last)
                def _():
                    d.start()
            else:
                @pl.when(is_last)
                def _():
                    d.start()

    def drain_stores(b, p, sem):
        for j, d in store_descs(b, p, sem):
            if j >= 0:
                @pl.when(not_last)
                def _():
                    d.wait()
            else:
                @pl.when(is_last)
                def _():
                    d.wait()

    def apply_math(p, j, sl, s, dx, dy, dw, dh):
        ax = ax_v[j, sl]
        ay = ay_v[j, sl]
        aw = aw_v[j, sl]
        ah = ah_v[j, sl]
        os_v[p, j, sl] = s
        ox_v[p, j, sl] = ax + dx * aw
        oy_v[p, j, sl] = ay + dy * ah
        ow_v[p, j, sl] = aw * jnp.exp(dw)
        oh_v[p, j, sl] = ah * jnp.exp(dh)

    def compute(p):
        @pl.when(span_ok)
        def _():
            for j in range(NJ):
                for i in range(G // 16):
                    sl = pl.ds(i * 16, 16)
                    loc = loc_v[j, sl]
                    apply_math(
                        p, j, sl,
                        plsc.load_gather(ws_v.at[p], [loc]),
                        plsc.load_gather(wdx_v.at[p], [loc]),
                        plsc.load_gather(wdy_v.at[p], [loc]),
                        plsc.load_gather(wdw_v.at[p], [loc]),
                        plsc.load_gather(wdh_v.at[p], [loc]))

        @pl.when(span_bad)
        def _():
            for j in range(NJ):
                for i in range(G // 16):
                    sl = pl.ds(i * 16, 16)
                    apply_math(
                        p, j, sl,
                        s_v[p, j, sl], dx_v[p, j, sl], dy_v[p, j, sl],
                        dw_v[p, j, sl], dh_v[p, j, sl])

    fire_gathers(0, 0, sem_ga)

    def body(t, carry):
        b0 = 2 * t
        b1 = 2 * t + 1
        fire_gathers(b1, 1, sem_gb)
        wait_gathers(b0, 0, sem_ga)

        @pl.when(t > 0)
        def _():
            drain_stores(b0 - 2, 0, sem_sa)
        compute(0)
        fire_stores(b0, 0, sem_sa)

        @pl.when(t < (B // 2) - 1)
        def _():
            fire_gathers(b0 + 2, 0, sem_ga)
        wait_gathers(b1, 1, sem_gb)

        @pl.when(t > 0)
        def _():
            drain_stores(b1 - 2, 1, sem_sb)
        compute(1)
        fire_stores(b1, 1, sem_sb)
        return carry

    lax.fori_loop(0, B // 2, body, 0)
    drain_stores(B - 2, 0, sem_sa)
    drain_stores(B - 1, 1, sem_sb)


_PLANE = pltpu.VMEM((NJ, G), jnp.float32)
_PLANE2 = pltpu.VMEM((2, NJ, G), jnp.float32)
_WIN2 = pltpu.VMEM((2, W), jnp.float32)


@functools.partial(
    pl.kernel,
    out_type=jax.ShapeDtypeStruct((5, B, V), jnp.float32),
    mesh=plsc.VectorSubcoreMesh(core_axis_name="c", subcore_axis_name="s"),
    compiler_params=pltpu.CompilerParams(
        needs_layout_passes=False, use_tc_tiling_on_sc=False),
    scratch_types=[
        pltpu.VMEM((NJ, G), jnp.int32),
        pltpu.VMEM((NJ, G), jnp.int32),
        _PLANE, _PLANE, _PLANE, _PLANE,
        _PLANE2, _PLANE2, _PLANE2, _PLANE2, _PLANE2,
        _WIN2, _WIN2, _WIN2, _WIN2, _WIN2,
        _PLANE2, _PLANE2, _PLANE2, _PLANE2, _PLANE2,
        pltpu.SemaphoreType.DMA,
        pltpu.SemaphoreType.DMA,
        pltpu.SemaphoreType.DMA,
        pltpu.SemaphoreType.DMA,
    ],
)
def _apply_deltas_sc(scores_hbm, deltas_hbm, anch_hbm, idx_hbm, out_hbm,
                     *refs):
    _sc_body(scores_hbm, deltas_hbm, anch_hbm, idx_hbm, out_hbm, *refs)


def kernel(scores, deltas, anchor_boxes, valid_indices):
    idx = valid_indices.astype(jnp.int32)
    idx = jnp.concatenate(
        [idx, jnp.broadcast_to(idx[-1], (VP - V,)).astype(jnp.int32)])
    deltas_t = jnp.transpose(deltas, (0, 2, 1)).reshape(B * 4, AB)
    anch_t = jnp.transpose(anchor_boxes, (1, 0))
    out = _apply_deltas_sc(scores, deltas_t, anch_t, idx)
    return jnp.transpose(out, (1, 2, 0))


# speedup vs baseline: 6.0078x; 1.0586x over previous
import functools

import jax
import jax.numpy as jnp
from jax import lax
from jax.experimental import pallas as pl
from jax.experimental.pallas import tpu as pltpu
from jax.experimental.pallas import tpu_sc as plsc

B = 16
AB = 20000
V = 12000
NW = 32
NJ = 3
G = 128
N = NJ * G
VP = NW * N
LASTN = V - (NW - 1) * N
W = 1024


def _sc_body(scores_hbm, deltas_hbm, anch_hbm, idx_hbm, out_hbm,
             idx_v, loc_v, ax_v, ay_v, aw_v, ah_v,
             s_v, dx_v, dy_v, dw_v, dh_v,
             ws_v, wd_v,
             os_v, ox_v, oy_v, ow_v, oh_v,
             sem_ga, sem_gb, sem_sa, sem_sb):
    wid = lax.axis_index("s") * 2 + lax.axis_index("c")
    base = wid * N
    is_last = wid == NW - 1
    not_last = wid != NW - 1

    for j in range(NJ):
        pltpu.sync_copy(idx_hbm.at[pl.ds(base + G * j, G)], idx_v.at[j])
    cps = []
    for j in range(NJ):
        ij = idx_v.at[j]
        cps.append(pltpu.async_copy(anch_hbm.at[0].at[ij], ax_v.at[j], sem_ga))
        cps.append(pltpu.async_copy(anch_hbm.at[1].at[ij], ay_v.at[j], sem_ga))
        cps.append(pltpu.async_copy(anch_hbm.at[2].at[ij], aw_v.at[j], sem_ga))
        cps.append(pltpu.async_copy(anch_hbm.at[3].at[ij], ah_v.at[j], sem_ga))
    for cp in cps:
        cp.wait()

    lo = lax.reduce_min(idx_v[0, pl.ds(0, 16)], (0,))
    hi = lax.reduce_max(idx_v[NJ - 1, pl.ds(G - 16, 16)], (0,))
    lo_al = pl.multiple_of(
        jnp.minimum((lo >> 3) << 3, jnp.int32(AB - W)), 8)
    span_ok = (hi - lo_al) < W
    span_bad = jnp.logical_not(span_ok)
    for j in range(NJ):
        for i in range(G // 16):
            sl = pl.ds(i * 16, 16)
            loc_v[j, sl] = idx_v[j, sl] - lo_al

    def fast_descs(b, p, sem):
        row = b * 4
        win = pl.ds(lo_al, W)
        return [
            pltpu.make_async_copy(scores_hbm.at[b].at[win], ws_v.at[p], sem),
            pltpu.make_async_copy(
                deltas_hbm.at[pl.ds(row, 4), win], wd_v.at[p], sem),
        ]

    def slow_descs(b, p, sem):
        row = b * 4
        ds = []
        for j in range(NJ):
            ij = idx_v.at[j]
            ds.append(pltpu.make_async_copy(
                scores_hbm.at[b].at[ij], s_v.at[p].at[j], sem))
            ds.append(pltpu.make_async_copy(
                deltas_hbm.at[row].at[ij], dx_v.at[p].at[j], sem))
            ds.append(pltpu.make_async_copy(
                deltas_hbm.at[row + 1].at[ij], dy_v.at[p].at[j], sem))
            ds.append(pltpu.make_async_copy(
                deltas_hbm.at[row + 2].at[ij], dw_v.at[p].at[j], sem))
            ds.append(pltpu.make_async_copy(
                deltas_hbm.at[row + 3].at[ij], dh_v.at[p].at[j], sem))
        return ds

    def fire_gathers(b, p, sem):
        @pl.when(span_ok)
        def _():
            for d in fast_descs(b, p, sem):
                d.start()

        @pl.when(span_bad)
        def _():
            for d in slow_descs(b, p, sem):
                d.start()

    def wait_gathers(b, p, sem):
        @pl.when(span_ok)
        def _():
            for d in fast_descs(b, p, sem):
                d.wait()

        @pl.when(span_bad)
        def _():
            for d in slow_descs(b, p, sem):
                d.wait()

    def store_descs(b, p, sem):
        ds = []
        for c, buf in ((0, os_v), (1, ox_v), (2, oy_v), (3, ow_v), (4, oh_v)):
            ds.append((0, pltpu.make_async_copy(
                buf.at[p],
                out_hbm.at[c].at[b].at[pl.ds(base, N)], sem)))
            ds.append((-1, pltpu.make_async_copy(
                buf.at[p].at[pl.ds(0, LASTN)],
                out_hbm.at[c].at[b].at[pl.ds(base, LASTN)], sem)))
        return ds

    def fire_stores(b, p, sem):
        for j, d in store_descs(b, p, sem):
            if j >= 0:
                @pl.when(not_last)
                def _():
                    d.start()
            else:
                @pl.when(is_last)
                def _():
                    d.start()

    def drain_stores(b, p, sem):
        for j, d in store_descs(b, p, sem):
            if j >= 0:
                @pl.when(not_last)
                def _():
                    d.wait()
            else:
                @pl.when(is_last)
                def _():
                    d.wait()

    def apply_math(p, j, sl, slo, s, dx, dy, dw, dh):
        ax = ax_v[j, sl]
        ay = ay_v[j, sl]
        aw = aw_v[j, sl]
        ah = ah_v[j, sl]
        os_v[p, slo] = s
        ox_v[p, slo] = ax + dx * aw
        oy_v[p, slo] = ay + dy * ah
        ow_v[p, slo] = aw * jnp.exp(dw)
        oh_v[p, slo] = ah * jnp.exp(dh)

    def compute(p):
        @pl.when(span_ok)
        def _():
            for j in range(NJ):
                for i in range(G // 16):
                    sl = pl.ds(i * 16, 16)
                    slo = pl.ds(j * G + i * 16, 16)
                    loc = loc_v[j, sl]
                    apply_math(
                        p, j, sl, slo,
                        plsc.load_gather(ws_v.at[p], [loc]),
                        plsc.load_gather(wd_v.at[p].at[0], [loc]),
                        plsc.load_gather(wd_v.at[p].at[1], [loc]),
                        plsc.load_gather(wd_v.at[p].at[2], [loc]),
                        plsc.load_gather(wd_v.at[p].at[3], [loc]))

        @pl.when(span_bad)
        def _():
            for j in range(NJ):
                for i in range(G // 16):
                    sl = pl.ds(i * 16, 16)
                    slo = pl.ds(j * G + i * 16, 16)
                    apply_math(
                        p, j, sl, slo,
                        s_v[p, j, sl], dx_v[p, j, sl], dy_v[p, j, sl],
                        dw_v[p, j, sl], dh_v[p, j, sl])

    fire_gathers(0, 0, sem_ga)

    def body(t, carry):
        b0 = 2 * t
        b1 = 2 * t + 1
        fire_gathers(b1, 1, sem_gb)
        wait_gathers(b0, 0, sem_ga)

        @pl.when(t > 0)
        def _():
            drain_stores(b0 - 2, 0, sem_sa)
        compute(0)
        fire_stores(b0, 0, sem_sa)

        @pl.when(t < (B // 2) - 1)
        def _():
            fire_gathers(b0 + 2, 0, sem_ga)
        wait_gathers(b1, 1, sem_gb)

        @pl.when(t > 0)
        def _():
            drain_stores(b1 - 2, 1, sem_sb)
        compute(1)
        fire_stores(b1, 1, sem_sb)
        return carry

    lax.fori_loop(0, B // 2, body, 0)
    drain_stores(B - 2, 0, sem_sa)
    drain_stores(B - 1, 1, sem_sb)


_PLANE = pltpu.VMEM((NJ, G), jnp.float32)
_PLANE2 = pltpu.VMEM((2, NJ, G), jnp.float32)
_FLAT2 = pltpu.VMEM((2, N), jnp.float32)


@functools.partial(
    pl.kernel,
    out_type=jax.ShapeDtypeStruct((5, B, V), jnp.float32),
    mesh=plsc.VectorSubcoreMesh(core_axis_name="c", subcore_axis_name="s"),
    compiler_params=pltpu.CompilerParams(
        needs_layout_passes=False, use_tc_tiling_on_sc=False),
    scratch_types=[
        pltpu.VMEM((NJ, G), jnp.int32),
        pltpu.VMEM((NJ, G), jnp.int32),
        _PLANE, _PLANE, _PLANE, _PLANE,
        _PLANE2, _PLANE2, _PLANE2, _PLANE2, _PLANE2,
        pltpu.VMEM((2, W), jnp.float32),
        pltpu.VMEM((2, 4, W), jnp.float32),
        _FLAT2, _FLAT2, _FLAT2, _FLAT2, _FLAT2,
        pltpu.SemaphoreType.DMA,
        pltpu.SemaphoreType.DMA,
        pltpu.SemaphoreType.DMA,
        pltpu.SemaphoreType.DMA,
    ],
)
def _apply_deltas_sc(scores_hbm, deltas_hbm, anch_hbm, idx_hbm, out_hbm,
                     *refs):
    _sc_body(scores_hbm, deltas_hbm, anch_hbm, idx_hbm, out_hbm, *refs)


def kernel(scores, deltas, anchor_boxes, valid_indices):
    idx = valid_indices.astype(jnp.int32)
    idx = jnp.concatenate(
        [idx, jnp.broadcast_to(idx[-1], (VP - V,)).astype(jnp.int32)])
    deltas_t = jnp.transpose(deltas, (0, 2, 1)).reshape(B * 4, AB)
    anch_t = jnp.transpose(anchor_boxes, (1, 0))
    out = _apply_deltas_sc(scores, deltas_t, anch_t, idx)
    return jnp.transpose(out, (1, 2, 0))
